# baseline (speedup 1.0000x reference)
import functools

import jax
import jax.numpy as jnp
from jax import lax
from jax.experimental import pallas as pl
from jax.experimental.pallas import tpu as pltpu
from jax.experimental.pallas import tpu_sc as plsc

_BN_EPS = 1e-5
_STATS_BLOCK = 4000
_MAIN_BLOCK = 4000
_WIN = 128


def _gather_rows_sc(table, idx):
    _, d = table.shape
    b = idx.shape[0]
    info = plsc.get_sparse_core_info()
    nw = info.num_cores * info.num_subcores
    b_per_w = b // nw
    mesh = plsc.VectorSubcoreMesh(core_axis_name="c", subcore_axis_name="s")

    @functools.partial(
        pl.kernel,
        mesh=mesh,
        out_type=jax.ShapeDtypeStruct((b, d), table.dtype),
        scratch_types=[
            pltpu.VMEM((b_per_w,), jnp.int32),
            pltpu.VMEM((b_per_w, d), table.dtype),
            pltpu.SemaphoreType.DMA,
        ],
    )
    def gather_kernel(table_hbm, idx_hbm, out_hbm, idx_v, rows_v, sem):
        wid = lax.axis_index("s") * info.num_cores + lax.axis_index("c")
        base = wid * b_per_w
        pltpu.sync_copy(idx_hbm.at[pl.ds(base, b_per_w)], idx_v)
        pltpu.async_copy(table_hbm.at[idx_v], rows_v, sem).wait()
        pltpu.sync_copy(rows_v, out_hbm.at[pl.ds(base, b_per_w)])

    return gather_kernel(table, idx)


def _stats_body(x_ref, o_ref):
    @pl.when(pl.program_id(0) == 0)
    def _init():
        o_ref[...] = jnp.zeros_like(o_ref)

    x = x_ref[...]
    s = jnp.sum(x, axis=0, keepdims=True)
    s2 = jnp.sum(x * x, axis=0, keepdims=True)
    pad = jnp.zeros((6, x.shape[1]), jnp.float32)
    o_ref[...] += jnp.concatenate([s, s2, pad], axis=0)


def _main_body(n_total, n_seg,
               x_ref, gid_ref, gid16_ref, stats_ref, fl_ref, wut_ref,
               wvt_ref, bv_ref, wet_ref, gamma_ref, beta_ref,
               o_ref, fv_ref, acc_ref):
    i = pl.program_id(0)
    nblocks = pl.num_programs(0)

    mean = stats_ref[0:1, :] * (1.0 / n_total)
    var = stats_ref[1:2, :] * (1.0 / n_total) - mean * mean
    rstd = lax.rsqrt(var + _BN_EPS)
    scale = rstd * gamma_ref[...]
    shift = beta_ref[...] - mean * scale

    @pl.when(i == 0)
    def _init():
        u0 = jnp.dot(shift, wut_ref[...].astype(jnp.float32),
                     preferred_element_type=jnp.float32)
        hl = fl_ref[...] * scale + shift
        fv_ref[0:n_seg, :] = (
            jnp.dot(hl, wvt_ref[...], preferred_element_type=jnp.float32)
            + bv_ref[...] + u0
        ).astype(jnp.bfloat16)
        fv_ref[n_seg:, :] = jnp.zeros((_WIN, fl_ref.shape[1]), jnp.bfloat16)
        acc_ref[...] = jnp.zeros_like(acc_ref)

    x = x_ref[...]
    t = x * scale
    xb = t.astype(jnp.bfloat16)
    h = t + shift
    g16 = gid16_ref[...]
    nb_rows = g16.shape[0]

    u = jnp.dot(xb, wut_ref[...], preferred_element_type=jnp.float32)

    def _attend(onehot, fv_blk):
        vb = jnp.dot(onehot, fv_blk, preferred_element_type=jnp.float32)
        arg = u + vb
        sgate = 1.0 / (1.0 + jnp.exp(-arg))
        e = jnp.dot(sgate, wet_ref[...], preferred_element_type=jnp.float32)
        w = jnp.exp(e)
        wb = jnp.broadcast_to(w.astype(jnp.bfloat16),
                              (nb_rows, x.shape[1]))
        hwb = (h * w).astype(jnp.bfloat16)
        return jnp.concatenate([hwb, wb], axis=1)

    g0 = gid_ref[0, 0]
    glast = gid_ref[nb_rows - 1, 0]
    base = pl.multiple_of((g0 // 16) * 16, 16)
    fits = glast - base < _WIN

    @pl.when(fits)
    def _window_path():
        segw = lax.broadcasted_iota(jnp.int16, (nb_rows, _WIN), 1)
        b16 = base.astype(jnp.int16)
        ohw = ((g16 - b16) == segw).astype(jnp.bfloat16)
        hw2 = _attend(ohw, fv_ref[pl.ds(base, _WIN), :])
        acc_ref[pl.ds(base, _WIN), :] += lax.dot_general(
            ohw, hw2, (((0,), (0,)), ((), ())),
            preferred_element_type=jnp.float32)

    @pl.when(jnp.logical_not(fits))
    def _full_path():
        seg = lax.broadcasted_iota(jnp.int16, (nb_rows, n_seg), 1)
        onehot = (g16 == seg).astype(jnp.bfloat16)
        hw2 = _attend(onehot, fv_ref[0:n_seg, :])
        acc_ref[0:n_seg, :] += lax.dot_general(
            onehot, hw2, (((0,), (0,)), ((), ())),
            preferred_element_type=jnp.float32)

    @pl.when(i == nblocks - 1)
    def _fin():
        d = x_ref.shape[1]
        aw = acc_ref[0:n_seg, d:d + 1]
        inv = jnp.where(aw > 0, 1.0 / aw, 0.0)
        o_ref[...] = acc_ref[0:n_seg, :d] * inv


def _pad_rows(a, nblk, fill):
    n = a.shape[0]
    npad = -(-n // nblk) * nblk
    if npad == n:
        return a
    return jnp.pad(a, ((0, npad - n),) + ((0, 0),) * (a.ndim - 1),
                   constant_values=fill)


def kernel(feat, graph_id, last_nodes, gamma, beta, Wu, Wv, bv, We):
    n, d = feat.shape
    b = last_nodes.shape[0]

    feat_last = _gather_rows_sc(feat, last_nodes.astype(jnp.int32))

    feat_s = _pad_rows(feat, _STATS_BLOCK, 0.0)
    nblk1 = feat_s.shape[0] // _STATS_BLOCK
    stats = pl.pallas_call(
        _stats_body,
        grid=(nblk1,),
        in_specs=[pl.BlockSpec((_STATS_BLOCK, d), lambda i: (i, 0))],
        out_specs=pl.BlockSpec((8, d), lambda i: (0, 0)),
        out_shape=jax.ShapeDtypeStruct((8, d), jnp.float32),
    )(feat_s)

    feat_m = _pad_rows(feat, _MAIN_BLOCK, 0.0)
    gid = _pad_rows(graph_id.astype(jnp.int32), _MAIN_BLOCK, b)
    gid = gid.reshape(-1, 1)
    gid16 = gid.astype(jnp.int16)
    nblk2 = feat_m.shape[0] // _MAIN_BLOCK

    full = lambda i: (0, 0)
    out = pl.pallas_call(
        functools.partial(_main_body, float(n), b),
        grid=(nblk2,),
        in_specs=[
            pl.BlockSpec((_MAIN_BLOCK, d), lambda i: (i, 0)),
            pl.BlockSpec((_MAIN_BLOCK, 1), lambda i: (i, 0)),
            pl.BlockSpec((_MAIN_BLOCK, 1), lambda i: (i, 0)),
            pl.BlockSpec((8, d), full),
            pl.BlockSpec((b, d), full),
            pl.BlockSpec((d, Wu.shape[0]), full),
            pl.BlockSpec((d, Wv.shape[0]), full),
            pl.BlockSpec((1, Wv.shape[0]), full),
            pl.BlockSpec((Wu.shape[0], 1), full),
            pl.BlockSpec((1, d), full),
            pl.BlockSpec((1, d), full),
        ],
        out_specs=pl.BlockSpec((b, d), full),
        out_shape=jax.ShapeDtypeStruct((b, d), jnp.float32),
        scratch_shapes=[
            pltpu.VMEM((b + _WIN, Wv.shape[0]), jnp.bfloat16),
            pltpu.VMEM((b + _WIN, 2 * d), jnp.float32),
        ],
    )(feat_m, gid, gid16, stats, feat_last, Wu.T.astype(jnp.bfloat16), Wv.T,
      bv.reshape(1, -1), We.T, gamma.reshape(1, -1), beta.reshape(1, -1))
    return out


# speedup vs baseline: 1.3243x; 1.3243x over previous
import functools

import jax
import jax.numpy as jnp
from jax import lax
from jax.experimental import pallas as pl
from jax.experimental.pallas import tpu as pltpu
from jax.experimental.pallas import tpu_sc as plsc

_BN_EPS = 1e-5
_STATS_BLOCK = 4000
_MAIN_BLOCK = 4000
_WIN = 128


def _gather_rows_sc(table, idx):
    _, d = table.shape
    b = idx.shape[0]
    info = plsc.get_sparse_core_info()
    nw = info.num_cores * info.num_subcores
    b_per_w = b // nw
    mesh = plsc.VectorSubcoreMesh(core_axis_name="c", subcore_axis_name="s")

    @functools.partial(
        pl.kernel,
        mesh=mesh,
        out_type=jax.ShapeDtypeStruct((b, d), table.dtype),
        scratch_types=[
            pltpu.VMEM((b_per_w,), jnp.int32),
            pltpu.VMEM((b_per_w, d), table.dtype),
            pltpu.SemaphoreType.DMA,
        ],
    )
    def gather_kernel(table_hbm, idx_hbm, out_hbm, idx_v, rows_v, sem):
        wid = lax.axis_index("s") * info.num_cores + lax.axis_index("c")
        base = wid * b_per_w
        pltpu.sync_copy(idx_hbm.at[pl.ds(base, b_per_w)], idx_v)
        pltpu.async_copy(table_hbm.at[idx_v], rows_v, sem).wait()
        pltpu.sync_copy(rows_v, out_hbm.at[pl.ds(base, b_per_w)])

    return gather_kernel(table, idx)


def _stats_body(x_ref, o_ref):
    @pl.when(pl.program_id(0) == 0)
    def _init():
        o_ref[...] = jnp.zeros_like(o_ref)

    x = x_ref[...]
    s = jnp.sum(x, axis=0, keepdims=True)
    s2 = jnp.sum(x * x, axis=0, keepdims=True)
    pad = jnp.zeros((6, x.shape[1]), jnp.float32)
    o_ref[...] += jnp.concatenate([s, s2, pad], axis=0)


def _main_body(n_total, n_seg,
               x_ref, gid_ref, stats_ref, fl_ref, wut_ref,
               wvt_ref, bv_ref, wet_ref, gamma_ref, beta_ref,
               o_ref, fv_ref, acc_ref):
    i = pl.program_id(0)
    nblocks = pl.num_programs(0)

    mean = stats_ref[0:1, :] * (1.0 / n_total)
    var = stats_ref[1:2, :] * (1.0 / n_total) - mean * mean
    rstd = lax.rsqrt(var + _BN_EPS)
    scale = rstd * gamma_ref[...]
    shift = beta_ref[...] - mean * scale

    @pl.when(i == 0)
    def _init():
        u0 = jnp.dot(shift, wut_ref[...].astype(jnp.float32),
                     preferred_element_type=jnp.float32)
        hl = fl_ref[...] * scale + shift
        fv_ref[0:n_seg, :] = (
            jnp.dot(hl, wvt_ref[...], preferred_element_type=jnp.float32)
            + bv_ref[...] + u0
        ).astype(jnp.bfloat16)
        fv_ref[n_seg:, :] = jnp.zeros((_WIN, fl_ref.shape[1]), jnp.bfloat16)
        acc_ref[...] = jnp.zeros_like(acc_ref)

    x = x_ref[...]
    t = x * scale
    xb = t.astype(jnp.bfloat16)
    h = t + shift
    g = gid_ref[...]
    nb_rows = g.shape[0]

    u = jnp.dot(xb, wut_ref[...], preferred_element_type=jnp.float32)

    def _attend(onehot, fv_blk):
        vb = jnp.dot(onehot, fv_blk, preferred_element_type=jnp.float32)
        arg = u + vb
        sgate = 1.0 / (1.0 + jnp.exp(-arg))
        e = jnp.dot(sgate, wet_ref[...], preferred_element_type=jnp.float32)
        w = jnp.exp(e)
        wb = jnp.broadcast_to(w.astype(jnp.bfloat16),
                              (nb_rows, x.shape[1]))
        hwb = (h * w).astype(jnp.bfloat16)
        return jnp.concatenate([hwb, wb], axis=1)

    g0 = gid_ref[0, 0]
    glast = gid_ref[nb_rows - 1, 0]
    base = pl.multiple_of((g0 // 16) * 16, 16)
    fits = glast - base < _WIN

    @pl.when(fits)
    def _window_path():
        segw = lax.broadcasted_iota(jnp.int32, (nb_rows, _WIN), 1)
        ohw = ((g - base) == segw).astype(jnp.bfloat16)
        hw2 = _attend(ohw, fv_ref[pl.ds(base, _WIN), :])
        acc_ref[pl.ds(base, _WIN), :] += lax.dot_general(
            ohw, hw2, (((0,), (0,)), ((), ())),
            preferred_element_type=jnp.float32)

    @pl.when(jnp.logical_not(fits))
    def _full_path():
        seg = lax.broadcasted_iota(jnp.int32, (nb_rows, n_seg), 1)
        onehot = (g == seg).astype(jnp.bfloat16)
        hw2 = _attend(onehot, fv_ref[0:n_seg, :])
        acc_ref[0:n_seg, :] += lax.dot_general(
            onehot, hw2, (((0,), (0,)), ((), ())),
            preferred_element_type=jnp.float32)

    @pl.when(i == nblocks - 1)
    def _fin():
        d = x_ref.shape[1]
        aw = acc_ref[0:n_seg, d:d + 1]
        inv = jnp.where(aw > 0, 1.0 / aw, 0.0)
        o_ref[...] = acc_ref[0:n_seg, :d] * inv


def _pad_rows(a, nblk, fill):
    n = a.shape[0]
    npad = -(-n // nblk) * nblk
    if npad == n:
        return a
    return jnp.pad(a, ((0, npad - n),) + ((0, 0),) * (a.ndim - 1),
                   constant_values=fill)


def kernel(feat, graph_id, last_nodes, gamma, beta, Wu, Wv, bv, We):
    n, d = feat.shape
    b = last_nodes.shape[0]

    feat_last = _gather_rows_sc(feat, last_nodes.astype(jnp.int32))

    feat_s = _pad_rows(feat, _STATS_BLOCK, 0.0)
    nblk1 = feat_s.shape[0] // _STATS_BLOCK
    stats = pl.pallas_call(
        _stats_body,
        grid=(nblk1,),
        in_specs=[pl.BlockSpec((_STATS_BLOCK, d), lambda i: (i, 0))],
        out_specs=pl.BlockSpec((8, d), lambda i: (0, 0)),
        out_shape=jax.ShapeDtypeStruct((8, d), jnp.float32),
    )(feat_s)

    feat_m = _pad_rows(feat, _MAIN_BLOCK, 0.0)
    gid = _pad_rows(graph_id.astype(jnp.int32), _MAIN_BLOCK, b)
    gid = gid.reshape(-1, 1)
    nblk2 = feat_m.shape[0] // _MAIN_BLOCK

    full = lambda i: (0, 0)
    out = pl.pallas_call(
        functools.partial(_main_body, float(n), b),
        grid=(nblk2,),
        in_specs=[
            pl.BlockSpec((_MAIN_BLOCK, d), lambda i: (i, 0)),
            pl.BlockSpec((_MAIN_BLOCK, 1), lambda i: (i, 0)),
            pl.BlockSpec((8, d), full),
            pl.BlockSpec((b, d), full),
            pl.BlockSpec((d, Wu.shape[0]), full),
            pl.BlockSpec((d, Wv.shape[0]), full),
            pl.BlockSpec((1, Wv.shape[0]), full),
            pl.BlockSpec((Wu.shape[0], 1), full),
            pl.BlockSpec((1, d), full),
            pl.BlockSpec((1, d), full),
        ],
        out_specs=pl.BlockSpec((b, d), full),
        out_shape=jax.ShapeDtypeStruct((b, d), jnp.float32),
        scratch_shapes=[
            pltpu.VMEM((b + _WIN, Wv.shape[0]), jnp.bfloat16),
            pltpu.VMEM((b + _WIN, 2 * d), jnp.float32),
        ],
    )(feat_m, gid, stats, feat_last, Wu.T.astype(jnp.bfloat16), Wv.T,
      bv.reshape(1, -1), We.T, gamma.reshape(1, -1), beta.reshape(1, -1))
    return out


# speedup vs baseline: 1.4257x; 1.0766x over previous
import functools

import jax
import jax.numpy as jnp
from jax import lax
from jax.experimental import pallas as pl
from jax.experimental.pallas import tpu as pltpu
from jax.experimental.pallas import tpu_sc as plsc

_BN_EPS = 1e-5
_MAIN_BLOCK = 4000
_WIN = 128


def _gather_rows_sc(table, idx):
    _, d = table.shape
    b = idx.shape[0]
    info = plsc.get_sparse_core_info()
    nw = info.num_cores * info.num_subcores
    b_per_w = b // nw
    mesh = plsc.VectorSubcoreMesh(core_axis_name="c", subcore_axis_name="s")

    @functools.partial(
        pl.kernel,
        mesh=mesh,
        out_type=jax.ShapeDtypeStruct((b, d), table.dtype),
        scratch_types=[
            pltpu.VMEM((b_per_w,), jnp.int32),
            pltpu.VMEM((b_per_w, d), table.dtype),
            pltpu.SemaphoreType.DMA,
        ],
    )
    def gather_kernel(table_hbm, idx_hbm, out_hbm, idx_v, rows_v, sem):
        wid = lax.axis_index("s") * info.num_cores + lax.axis_index("c")
        base = wid * b_per_w
        pltpu.sync_copy(idx_hbm.at[pl.ds(base, b_per_w)], idx_v)
        pltpu.async_copy(table_hbm.at[idx_v], rows_v, sem).wait()
        pltpu.sync_copy(rows_v, out_hbm.at[pl.ds(base, b_per_w)])

    return gather_kernel(table, idx)


def _fused_body(n_total, n_seg,
                x_ref, gid_ref, fl_ref, wu_ref, wv_ref,
                bv_ref, we_ref, gamma_ref, beta_ref,
                o_ref, stats_ref, fv_ref, acc_ref, wub_ref):
    p = pl.program_id(0)
    i = pl.program_id(1)
    nblocks = pl.num_programs(1)
    d = x_ref.shape[1]

    @pl.when(p == 0)
    def _phase_stats():
        @pl.when(i == 0)
        def _z():
            stats_ref[...] = jnp.zeros_like(stats_ref)

        x = x_ref[...]
        s = jnp.sum(x, axis=0, keepdims=True)
        s2 = jnp.sum(x * x, axis=0, keepdims=True)
        pad = jnp.zeros((6, d), jnp.float32)
        stats_ref[...] += jnp.concatenate([s, s2, pad], axis=0)

    @pl.when(p == 1)
    def _phase_main():
        mean = stats_ref[0:1, :] * (1.0 / n_total)
        var = stats_ref[1:2, :] * (1.0 / n_total) - mean * mean
        rstd = lax.rsqrt(var + _BN_EPS)
        scale = rstd * gamma_ref[...]
        shift = beta_ref[...] - mean * scale
        t_rhs = (((1,), (1,)), ((), ()))

        @pl.when(i == 0)
        def _init():
            u0 = lax.dot_general(shift, wu_ref[...], t_rhs,
                                 preferred_element_type=jnp.float32)
            hl = fl_ref[...] * scale + shift
            fv_ref[0:n_seg, :] = (
                lax.dot_general(hl, wv_ref[...], t_rhs,
                                preferred_element_type=jnp.float32)
                + bv_ref[...] + u0
            ).astype(jnp.bfloat16)
            fv_ref[n_seg:, :] = jnp.zeros((_WIN, fv_ref.shape[1]),
                                          jnp.bfloat16)
            acc_ref[...] = jnp.zeros_like(acc_ref)
            wub_ref[...] = wu_ref[...].astype(jnp.bfloat16)

        x = x_ref[...]
        t = x * scale
        xb = t.astype(jnp.bfloat16)
        h = t + shift
        g = gid_ref[...]
        nb_rows = g.shape[0]

        u = lax.dot_general(xb, wub_ref[...], t_rhs,
                            preferred_element_type=jnp.float32)

        def _attend(onehot, fv_blk):
            vb = jnp.dot(onehot, fv_blk, preferred_element_type=jnp.float32)
            arg = u + vb
            sgate = 1.0 / (1.0 + jnp.exp(-arg))
            e = lax.dot_general(sgate, we_ref[...], t_rhs,
                                preferred_element_type=jnp.float32)
            w = jnp.exp(e)
            wb = jnp.broadcast_to(w.astype(jnp.bfloat16), (nb_rows, d))
            hwb = (h * w).astype(jnp.bfloat16)
            return jnp.concatenate([hwb, wb], axis=1)

        g0 = gid_ref[0, 0]
        glast = gid_ref[nb_rows - 1, 0]
        base = pl.multiple_of((g0 // 16) * 16, 16)
        fits = glast - base < _WIN

        @pl.when(fits)
        def _window_path():
            segw = lax.broadcasted_iota(jnp.int32, (nb_rows, _WIN), 1)
            ohw = ((g - base) == segw).astype(jnp.bfloat16)
            hw2 = _attend(ohw, fv_ref[pl.ds(base, _WIN), :])
            acc_ref[pl.ds(base, _WIN), :] += lax.dot_general(
                ohw, hw2, (((0,), (0,)), ((), ())),
                preferred_element_type=jnp.float32)

        @pl.when(jnp.logical_not(fits))
        def _full_path():
            seg = lax.broadcasted_iota(jnp.int32, (nb_rows, n_seg), 1)
            onehot = (g == seg).astype(jnp.bfloat16)
            hw2 = _attend(onehot, fv_ref[0:n_seg, :])
            acc_ref[0:n_seg, :] += lax.dot_general(
                onehot, hw2, (((0,), (0,)), ((), ())),
                preferred_element_type=jnp.float32)

        @pl.when(i == nblocks - 1)
        def _fin():
            aw = acc_ref[0:n_seg, d:d + 1]
            inv = jnp.where(aw > 0, 1.0 / aw, 0.0)
            o_ref[...] = acc_ref[0:n_seg, :d] * inv


def _pad_rows(a, nblk, fill):
    n = a.shape[0]
    npad = -(-n // nblk) * nblk
    if npad == n:
        return a
    return jnp.pad(a, ((0, npad - n),) + ((0, 0),) * (a.ndim - 1),
                   constant_values=fill)


def kernel(feat, graph_id, last_nodes, gamma, beta, Wu, Wv, bv, We):
    n, d = feat.shape
    b = last_nodes.shape[0]
    h_dim = Wu.shape[0]

    feat_last = _gather_rows_sc(feat, last_nodes.astype(jnp.int32))

    feat_m = _pad_rows(feat, _MAIN_BLOCK, 0.0)
    gid = _pad_rows(graph_id.astype(jnp.int32), _MAIN_BLOCK, b)
    gid = gid.reshape(-1, 1)
    nblk = feat_m.shape[0] // _MAIN_BLOCK

    full = lambda p, i: (0, 0)
    out = pl.pallas_call(
        functools.partial(_fused_body, float(n), b),
        grid=(2, nblk),
        in_specs=[
            pl.BlockSpec((_MAIN_BLOCK, d), lambda p, i: (i, 0)),
            pl.BlockSpec((_MAIN_BLOCK, 1), lambda p, i: (i, 0)),
            pl.BlockSpec((b, d), full),
            pl.BlockSpec((h_dim, d), full),
            pl.BlockSpec((h_dim, d), full),
            pl.BlockSpec((1, h_dim), full),
            pl.BlockSpec((1, h_dim), full),
            pl.BlockSpec((1, d), full),
            pl.BlockSpec((1, d), full),
        ],
        out_specs=pl.BlockSpec((b, d), full),
        out_shape=jax.ShapeDtypeStruct((b, d), jnp.float32),
        scratch_shapes=[
            pltpu.VMEM((8, d), jnp.float32),
            pltpu.VMEM((b + _WIN, h_dim), jnp.bfloat16),
            pltpu.VMEM((b + _WIN, 2 * d), jnp.float32),
            pltpu.VMEM((h_dim, d), jnp.bfloat16),
        ],
    )(feat_m, gid, feat_last, Wu, Wv,
      bv.reshape(1, -1), We, gamma.reshape(1, -1), beta.reshape(1, -1))
    return out


# speedup vs baseline: 2.2695x; 1.5918x over previous
import functools

import jax
import jax.numpy as jnp
from jax import lax
from jax.experimental import pallas as pl
from jax.experimental.pallas import tpu as pltpu
from jax.experimental.pallas import tpu_sc as plsc

_BN_EPS = 1e-5
_MAIN_BLOCK = 4000
_WIN = 128


def _gather_rows_sc(table, idx):
    _, d = table.shape
    b = idx.shape[0]
    info = plsc.get_sparse_core_info()
    nw = info.num_cores * info.num_subcores
    b_per_w = b // nw
    mesh = plsc.VectorSubcoreMesh(core_axis_name="c", subcore_axis_name="s")

    @functools.partial(
        pl.kernel,
        mesh=mesh,
        out_type=jax.ShapeDtypeStruct((b, d), table.dtype),
        scratch_types=[
            pltpu.VMEM((b_per_w,), jnp.int32),
            pltpu.VMEM((b_per_w, d), table.dtype),
            pltpu.SemaphoreType.DMA,
        ],
    )
    def gather_kernel(table_hbm, idx_hbm, out_hbm, idx_v, rows_v, sem):
        wid = lax.axis_index("s") * info.num_cores + lax.axis_index("c")
        base = wid * b_per_w
        pltpu.sync_copy(idx_hbm.at[pl.ds(base, b_per_w)], idx_v)
        pltpu.async_copy(table_hbm.at[idx_v], rows_v, sem).wait()
        pltpu.sync_copy(rows_v, out_hbm.at[pl.ds(base, b_per_w)])

    return gather_kernel(table, idx)


def _fused_body(n_total, n_seg,
                x_ref, gid_ref, fl_ref, wu_ref, wv_ref,
                bv_ref, we_ref, gamma_ref, beta_ref,
                o_ref, stats_ref, fv_ref, acc_ref, wub_ref):
    p = pl.program_id(0)
    i = pl.program_id(1)
    nblocks = pl.num_programs(1)
    d = x_ref.shape[1]

    @pl.when(p == 0)
    def _phase_stats():
        @pl.when(i == 0)
        def _z():
            stats_ref[...] = jnp.zeros_like(stats_ref)

        x = x_ref[...]
        s = jnp.sum(x, axis=0, keepdims=True)
        s2 = jnp.sum(x * x, axis=0, keepdims=True)
        pad = jnp.zeros((6, d), jnp.float32)
        stats_ref[...] += jnp.concatenate([s, s2, pad], axis=0)

    @pl.when(p == 1)
    def _phase_main():
        mean = stats_ref[0:1, :] * (1.0 / n_total)
        var = stats_ref[1:2, :] * (1.0 / n_total) - mean * mean
        rstd = lax.rsqrt(var + _BN_EPS)
        scale = rstd * gamma_ref[...]
        shift = beta_ref[...] - mean * scale
        t_rhs = (((1,), (1,)), ((), ()))

        @pl.when(i == 0)
        def _init():
            u0 = lax.dot_general(shift, wu_ref[...], t_rhs,
                                 preferred_element_type=jnp.float32)
            hl = fl_ref[...] * scale + shift
            fv_ref[0:n_seg, :] = (
                lax.dot_general(hl, wv_ref[...], t_rhs,
                                preferred_element_type=jnp.float32)
                + bv_ref[...] + u0
            ).astype(jnp.bfloat16)
            fv_ref[n_seg:, :] = jnp.zeros((_WIN, fv_ref.shape[1]),
                                          jnp.bfloat16)
            acc_ref[...] = jnp.zeros_like(acc_ref)
            wub_ref[...] = wu_ref[...].astype(jnp.bfloat16)

        x = x_ref[...]
        t = x * scale
        xb = t.astype(jnp.bfloat16)
        h = t + shift
        g_row = gid_ref[0]
        nb_rows = x.shape[0]

        u = lax.dot_general(xb, wub_ref[...], t_rhs,
                            preferred_element_type=jnp.float32)

        def _attend(oh_t, fv_blk):
            vb = lax.dot_general(oh_t, fv_blk, (((0,), (0,)), ((), ())),
                                 preferred_element_type=jnp.float32)
            arg = u + vb
            sgate = 1.0 / (1.0 + jnp.exp(-arg))
            e = lax.dot_general(sgate, we_ref[...], t_rhs,
                                preferred_element_type=jnp.float32)
            w = jnp.exp(e)
            wb = jnp.broadcast_to(w.astype(jnp.bfloat16), (nb_rows, d))
            hwb = (h * w).astype(jnp.bfloat16)
            return jnp.concatenate([hwb, wb], axis=1)

        g0 = gid_ref[0, 0, 0]
        glast = gid_ref[0, 0, nb_rows - 1]
        base = pl.multiple_of((g0 // 16) * 16, 16)
        fits = glast - base < _WIN

        @pl.when(fits)
        def _window_path():
            segw = lax.broadcasted_iota(jnp.int32, (_WIN, nb_rows), 0)
            oh_t = ((g_row - base) == segw).astype(jnp.bfloat16)
            hw2 = _attend(oh_t, fv_ref[pl.ds(base, _WIN), :])
            acc_ref[pl.ds(base, _WIN), :] += jnp.dot(
                oh_t, hw2, preferred_element_type=jnp.float32)

        @pl.when(jnp.logical_not(fits))
        def _full_path():
            seg = lax.broadcasted_iota(jnp.int32, (n_seg, nb_rows), 0)
            oh_t = (g_row == seg).astype(jnp.bfloat16)
            hw2 = _attend(oh_t, fv_ref[0:n_seg, :])
            acc_ref[0:n_seg, :] += jnp.dot(
                oh_t, hw2, preferred_element_type=jnp.float32)

        @pl.when(i == nblocks - 1)
        def _fin():
            aw = acc_ref[0:n_seg, d:d + 1]
            inv = jnp.where(aw > 0, 1.0 / aw, 0.0)
            o_ref[...] = acc_ref[0:n_seg, :d] * inv


def _pad_rows(a, nblk, fill):
    n = a.shape[0]
    npad = -(-n // nblk) * nblk
    if npad == n:
        return a
    return jnp.pad(a, ((0, npad - n),) + ((0, 0),) * (a.ndim - 1),
                   constant_values=fill)


def kernel(feat, graph_id, last_nodes, gamma, beta, Wu, Wv, bv, We):
    n, d = feat.shape
    b = last_nodes.shape[0]
    h_dim = Wu.shape[0]

    feat_last = _gather_rows_sc(feat, last_nodes.astype(jnp.int32))

    feat_m = _pad_rows(feat, _MAIN_BLOCK, 0.0)
    gid = _pad_rows(graph_id.astype(jnp.int32), _MAIN_BLOCK, b)
    gid = gid.reshape(-1, 1, _MAIN_BLOCK)
    nblk = feat_m.shape[0] // _MAIN_BLOCK

    full = lambda p, i: (0, 0)
    out = pl.pallas_call(
        functools.partial(_fused_body, float(n), b),
        grid=(2, nblk),
        in_specs=[
            pl.BlockSpec((_MAIN_BLOCK, d), lambda p, i: (i, 0)),
            pl.BlockSpec((1, 1, _MAIN_BLOCK),
                         lambda p, i: (i * p, 0, 0)),
            pl.BlockSpec((b, d), full),
            pl.BlockSpec((h_dim, d), full),
            pl.BlockSpec((h_dim, d), full),
            pl.BlockSpec((1, h_dim), full),
            pl.BlockSpec((1, h_dim), full),
            pl.BlockSpec((1, d), full),
            pl.BlockSpec((1, d), full),
        ],
        out_specs=pl.BlockSpec((b, d), full),
        out_shape=jax.ShapeDtypeStruct((b, d), jnp.float32),
        scratch_shapes=[
            pltpu.VMEM((8, d), jnp.float32),
            pltpu.VMEM((b + _WIN, h_dim), jnp.bfloat16),
            pltpu.VMEM((b + _WIN, 2 * d), jnp.float32),
            pltpu.VMEM((h_dim, d), jnp.bfloat16),
        ],
    )(feat_m, gid, feat_last, Wu, Wv,
      bv.reshape(1, -1), We, gamma.reshape(1, -1), beta.reshape(1, -1))
    return out


# speedup vs baseline: 2.4302x; 1.0708x over previous
import functools

import jax
import jax.numpy as jnp
from jax import lax
from jax.experimental import pallas as pl
from jax.experimental.pallas import tpu as pltpu
from jax.experimental.pallas import tpu_sc as plsc

_BN_EPS = 1e-5
_MAIN_BLOCK = 4000
_WIN = 128


def _gather_rows_sc(table, idx):
    _, d = table.shape
    b = idx.shape[0]
    info = plsc.get_sparse_core_info()
    nw = info.num_cores * info.num_subcores
    b_per_w = b // nw
    mesh = plsc.VectorSubcoreMesh(core_axis_name="c", subcore_axis_name="s")

    @functools.partial(
        pl.kernel,
        mesh=mesh,
        out_type=jax.ShapeDtypeStruct((b, d), table.dtype),
        scratch_types=[
            pltpu.VMEM((b_per_w,), jnp.int32),
            pltpu.VMEM((b_per_w, d), table.dtype),
            pltpu.SemaphoreType.DMA,
        ],
    )
    def gather_kernel(table_hbm, idx_hbm, out_hbm, idx_v, rows_v, sem):
        wid = lax.axis_index("s") * info.num_cores + lax.axis_index("c")
        base = wid * b_per_w
        pltpu.sync_copy(idx_hbm.at[pl.ds(base, b_per_w)], idx_v)
        pltpu.async_copy(table_hbm.at[idx_v], rows_v, sem).wait()
        pltpu.sync_copy(rows_v, out_hbm.at[pl.ds(base, b_per_w)])

    return gather_kernel(table, idx)


_GCHUNK = 128


def _fused_body(n_total, n_seg,
                x_ref, gid_ref, feat_any, ln_ref, wu_ref, wv_ref,
                bv_ref, we_ref, gamma_ref, beta_ref,
                o_ref, stats_ref, fv_ref, acc_ref, wub_ref, fl_ref, gsem):
    p = pl.program_id(0)
    i = pl.program_id(1)
    nblocks = pl.num_programs(1)
    d = x_ref.shape[1]
    nchunks = n_seg // _GCHUNK

    def _row_copy(k):
        row = ln_ref[0, k]
        return pltpu.make_async_copy(
            feat_any.at[pl.ds(row, 1), :],
            fl_ref.at[pl.ds(k, 1), :],
            gsem)

    @pl.when(p == 0)
    def _phase_stats():
        @pl.when(i == 0)
        def _z():
            stats_ref[...] = jnp.zeros_like(stats_ref)

        @pl.when(i < nchunks)
        def _issue():
            def body(j, c):
                _row_copy(i * _GCHUNK + j).start()
                return c
            lax.fori_loop(0, _GCHUNK, body, 0)

        @pl.when(jnp.logical_and(i >= 2, i < nchunks + 2))
        def _drain():
            def body(j, c):
                _row_copy((i - 2) * _GCHUNK + j).wait()
                return c
            lax.fori_loop(0, _GCHUNK, body, 0)

        x = x_ref[...]
        s = jnp.sum(x, axis=0, keepdims=True)
        s2 = jnp.sum(x * x, axis=0, keepdims=True)
        pad = jnp.zeros((6, d), jnp.float32)
        stats_ref[...] += jnp.concatenate([s, s2, pad], axis=0)

    @pl.when(p == 1)
    def _phase_main():
        mean = stats_ref[0:1, :] * (1.0 / n_total)
        var = stats_ref[1:2, :] * (1.0 / n_total) - mean * mean
        rstd = lax.rsqrt(var + _BN_EPS)
        scale = rstd * gamma_ref[...]
        shift = beta_ref[...] - mean * scale
        t_rhs = (((1,), (1,)), ((), ()))

        @pl.when(i == 0)
        def _init():
            u0 = lax.dot_general(shift, wu_ref[...], t_rhs,
                                 preferred_element_type=jnp.float32)
            hl = fl_ref[...] * scale + shift
            fv_ref[0:n_seg, :] = (
                lax.dot_general(hl, wv_ref[...], t_rhs,
                                preferred_element_type=jnp.float32)
                + bv_ref[...] + u0
            ).astype(jnp.bfloat16)
            fv_ref[n_seg:, :] = jnp.zeros((_WIN, fv_ref.shape[1]),
                                          jnp.bfloat16)
            acc_ref[...] = jnp.zeros_like(acc_ref)
            wub_ref[...] = wu_ref[...].astype(jnp.bfloat16)

        x = x_ref[...]
        t = x * scale
        xb = t.astype(jnp.bfloat16)
        h = t + shift
        g_row = gid_ref[0]
        nb_rows = x.shape[0]

        u = lax.dot_general(xb, wub_ref[...], t_rhs,
                            preferred_element_type=jnp.float32)

        def _attend(oh_t, fv_blk):
            vb = lax.dot_general(oh_t, fv_blk, (((0,), (0,)), ((), ())),
                                 preferred_element_type=jnp.float32)
            arg = u + vb
            sgate = 1.0 / (1.0 + jnp.exp(-arg))
            e = lax.dot_general(sgate, we_ref[...], t_rhs,
                                preferred_element_type=jnp.float32)
            w = jnp.exp(e)
            wb = jnp.broadcast_to(w.astype(jnp.bfloat16), (nb_rows, d))
            hwb = (h * w).astype(jnp.bfloat16)
            return jnp.concatenate([hwb, wb], axis=1)

        g0 = gid_ref[0, 0, 0]
        glast = gid_ref[0, 0, nb_rows - 1]
        base = pl.multiple_of((g0 // 16) * 16, 16)
        fits = glast - base < _WIN

        @pl.when(fits)
        def _window_path():
            segw = lax.broadcasted_iota(jnp.int32, (_WIN, nb_rows), 0)
            oh_t = ((g_row - base) == segw).astype(jnp.bfloat16)
            hw2 = _attend(oh_t, fv_ref[pl.ds(base, _WIN), :])
            acc_ref[pl.ds(base, _WIN), :] += jnp.dot(
                oh_t, hw2, preferred_element_type=jnp.float32)

        @pl.when(jnp.logical_not(fits))
        def _full_path():
            seg = lax.broadcasted_iota(jnp.int32, (n_seg, nb_rows), 0)
            oh_t = (g_row == seg).astype(jnp.bfloat16)
            hw2 = _attend(oh_t, fv_ref[0:n_seg, :])
            acc_ref[0:n_seg, :] += jnp.dot(
                oh_t, hw2, preferred_element_type=jnp.float32)

        @pl.when(i == nblocks - 1)
        def _fin():
            aw = acc_ref[0:n_seg, d:d + 1]
            inv = jnp.where(aw > 0, 1.0 / aw, 0.0)
            o_ref[...] = acc_ref[0:n_seg, :d] * inv


def _pad_rows(a, nblk, fill):
    n = a.shape[0]
    npad = -(-n // nblk) * nblk
    if npad == n:
        return a
    return jnp.pad(a, ((0, npad - n),) + ((0, 0),) * (a.ndim - 1),
                   constant_values=fill)


def kernel(feat, graph_id, last_nodes, gamma, beta, Wu, Wv, bv, We):
    n, d = feat.shape
    b = last_nodes.shape[0]
    h_dim = Wu.shape[0]

    ln2 = last_nodes.astype(jnp.int32).reshape(1, b)

    feat_m = _pad_rows(feat, _MAIN_BLOCK, 0.0)
    gid = _pad_rows(graph_id.astype(jnp.int32), _MAIN_BLOCK, b)
    gid = gid.reshape(-1, 1, _MAIN_BLOCK)
    nblk = feat_m.shape[0] // _MAIN_BLOCK

    full = lambda p, i: (0, 0)
    out = pl.pallas_call(
        functools.partial(_fused_body, float(n), b),
        grid=(2, nblk),
        in_specs=[
            pl.BlockSpec((_MAIN_BLOCK, d), lambda p, i: (i, 0)),
            pl.BlockSpec((1, 1, _MAIN_BLOCK),
                         lambda p, i: (i * p, 0, 0)),
            pl.BlockSpec(memory_space=pltpu.MemorySpace.HBM),
            pl.BlockSpec(memory_space=pltpu.MemorySpace.SMEM),
            pl.BlockSpec((h_dim, d), full),
            pl.BlockSpec((h_dim, d), full),
            pl.BlockSpec((1, h_dim), full),
            pl.BlockSpec((1, h_dim), full),
            pl.BlockSpec((1, d), full),
            pl.BlockSpec((1, d), full),
        ],
        out_specs=pl.BlockSpec((b, d), full),
        out_shape=jax.ShapeDtypeStruct((b, d), jnp.float32),
        scratch_shapes=[
            pltpu.VMEM((8, d), jnp.float32),
            pltpu.VMEM((b + _WIN, h_dim), jnp.bfloat16),
            pltpu.VMEM((b + _WIN, 2 * d), jnp.float32),
            pltpu.VMEM((h_dim, d), jnp.bfloat16),
            pltpu.VMEM((b, d), jnp.float32),
            pltpu.SemaphoreType.DMA,
        ],
    )(feat_m, gid, feat_m, ln2, Wu, Wv,
      bv.reshape(1, -1), We, gamma.reshape(1, -1), beta.reshape(1, -1))
    return out


# speedup vs baseline: 2.7251x; 1.1213x over previous
import functools

import jax
import jax.numpy as jnp
from jax import lax
from jax.experimental import pallas as pl
from jax.experimental.pallas import tpu as pltpu
from jax.experimental.pallas import tpu_sc as plsc

_BN_EPS = 1e-5
_MAIN_BLOCK = 4000
_WIN = 128


def _gather_rows_sc(table, idx):
    _, d = table.shape
    b = idx.shape[0]
    info = plsc.get_sparse_core_info()
    nw = info.num_cores * info.num_subcores
    b_per_w = b // nw
    mesh = plsc.VectorSubcoreMesh(core_axis_name="c", subcore_axis_name="s")

    @functools.partial(
        pl.kernel,
        mesh=mesh,
        out_type=jax.ShapeDtypeStruct((b, d), table.dtype),
        scratch_types=[
            pltpu.VMEM((b_per_w,), jnp.int32),
            pltpu.VMEM((b_per_w, d), table.dtype),
            pltpu.SemaphoreType.DMA,
        ],
    )
    def gather_kernel(table_hbm, idx_hbm, out_hbm, idx_v, rows_v, sem):
        wid = lax.axis_index("s") * info.num_cores + lax.axis_index("c")
        base = wid * b_per_w
        pltpu.sync_copy(idx_hbm.at[pl.ds(base, b_per_w)], idx_v)
        pltpu.async_copy(table_hbm.at[idx_v], rows_v, sem).wait()
        pltpu.sync_copy(rows_v, out_hbm.at[pl.ds(base, b_per_w)])

    return gather_kernel(table, idx)


_GCHUNK = 128


def _fused_body(n_total, n_seg,
                x_ref, gid_ref, feat_any, ln_ref, wu_ref, wv_ref,
                bv_ref, we_ref, gamma_ref, beta_ref,
                o_ref, stats_ref, fv_ref, acc_ref, wub_ref, fl_ref, gsem):
    p = pl.program_id(0)
    i = pl.program_id(1)
    nblocks = pl.num_programs(1)
    d = x_ref.shape[1]
    nchunks = n_seg // _GCHUNK

    def _row_copy(k):
        row = ln_ref[0, k]
        return pltpu.make_async_copy(
            feat_any.at[pl.ds(row, 1), :],
            fl_ref.at[pl.ds(k, 1), :],
            gsem)

    @pl.when(p == 0)
    def _phase_stats():
        @pl.when(i == 0)
        def _z():
            stats_ref[...] = jnp.zeros_like(stats_ref)

        @pl.when(i < nchunks)
        def _issue():
            def body(j, c):
                _row_copy(i * _GCHUNK + j).start()
                return c
            lax.fori_loop(0, _GCHUNK, body, 0, unroll=16)

        @pl.when(jnp.logical_and(i >= 2, i < nchunks + 2))
        def _drain():
            def body(j, c):
                _row_copy((i - 2) * _GCHUNK + j).wait()
                return c
            lax.fori_loop(0, _GCHUNK, body, 0, unroll=16)

        x = x_ref[...]
        s = jnp.sum(x, axis=0, keepdims=True)
        s2 = jnp.sum(x * x, axis=0, keepdims=True)
        pad = jnp.zeros((6, d), jnp.float32)
        stats_ref[...] += jnp.concatenate([s, s2, pad], axis=0)

    @pl.when(p == 1)
    def _phase_main():
        mean = stats_ref[0:1, :] * (1.0 / n_total)
        var = stats_ref[1:2, :] * (1.0 / n_total) - mean * mean
        rstd = lax.rsqrt(var + _BN_EPS)
        scale = rstd * gamma_ref[...]
        shift = beta_ref[...] - mean * scale
        t_rhs = (((1,), (1,)), ((), ()))

        @pl.when(i == 0)
        def _init():
            u0 = lax.dot_general(shift, wu_ref[...], t_rhs,
                                 preferred_element_type=jnp.float32)
            hl = fl_ref[...] * scale + shift
            fv_ref[0:n_seg, :] = (
                lax.dot_general(hl, wv_ref[...], t_rhs,
                                preferred_element_type=jnp.float32)
                + bv_ref[...] + u0
            ).astype(jnp.bfloat16)
            fv_ref[n_seg:, :] = jnp.zeros((_WIN, fv_ref.shape[1]),
                                          jnp.bfloat16)
            acc_ref[...] = jnp.zeros_like(acc_ref)
            wub_ref[...] = wu_ref[...].astype(jnp.bfloat16)

        x = x_ref[...]
        t = x * scale
        xb = t.astype(jnp.bfloat16)
        h = t + shift
        g_row = gid_ref[0]
        nb_rows = x.shape[0]

        u = lax.dot_general(xb, wub_ref[...], t_rhs,
                            preferred_element_type=jnp.float32)

        def _attend(oh_t, fv_blk):
            vb = lax.dot_general(oh_t, fv_blk, (((0,), (0,)), ((), ())),
                                 preferred_element_type=jnp.float32)
            arg = u + vb
            sgate = 1.0 / (1.0 + jnp.exp(-arg))
            e = lax.dot_general(sgate, we_ref[...], t_rhs,
                                preferred_element_type=jnp.float32)
            w = jnp.exp(e)
            wb = jnp.broadcast_to(w.astype(jnp.bfloat16), (nb_rows, d))
            hwb = (h * w).astype(jnp.bfloat16)
            return jnp.concatenate([hwb, wb], axis=1)

        g0 = gid_ref[0, 0, 0]
        glast = gid_ref[0, 0, nb_rows - 1]
        base = pl.multiple_of((g0 // 16) * 16, 16)
        fits = glast - base < _WIN

        @pl.when(fits)
        def _window_path():
            segw = lax.broadcasted_iota(jnp.int32, (_WIN, nb_rows), 0)
            oh_t = ((g_row - base) == segw).astype(jnp.bfloat16)
            hw2 = _attend(oh_t, fv_ref[pl.ds(base, _WIN), :])
            acc_ref[pl.ds(base, _WIN), :] += jnp.dot(
                oh_t, hw2, preferred_element_type=jnp.float32)

        @pl.when(jnp.logical_not(fits))
        def _full_path():
            seg = lax.broadcasted_iota(jnp.int32, (n_seg, nb_rows), 0)
            oh_t = (g_row == seg).astype(jnp.bfloat16)
            hw2 = _attend(oh_t, fv_ref[0:n_seg, :])
            acc_ref[0:n_seg, :] += jnp.dot(
                oh_t, hw2, preferred_element_type=jnp.float32)

        @pl.when(i == nblocks - 1)
        def _fin():
            aw = acc_ref[0:n_seg, d:d + 1]
            inv = jnp.where(aw > 0, 1.0 / aw, 0.0)
            o_ref[...] = acc_ref[0:n_seg, :d] * inv


def _pad_rows(a, nblk, fill):
    n = a.shape[0]
    npad = -(-n // nblk) * nblk
    if npad == n:
        return a
    return jnp.pad(a, ((0, npad - n),) + ((0, 0),) * (a.ndim - 1),
                   constant_values=fill)


def kernel(feat, graph_id, last_nodes, gamma, beta, Wu, Wv, bv, We):
    n, d = feat.shape
    b = last_nodes.shape[0]
    h_dim = Wu.shape[0]

    ln2 = last_nodes.astype(jnp.int32).reshape(1, b)

    feat_m = _pad_rows(feat, _MAIN_BLOCK, 0.0)
    gid = _pad_rows(graph_id.astype(jnp.int32), _MAIN_BLOCK, b)
    gid = gid.reshape(-1, 1, _MAIN_BLOCK)
    nblk = feat_m.shape[0] // _MAIN_BLOCK

    full = lambda p, i: (0, 0)
    out = pl.pallas_call(
        functools.partial(_fused_body, float(n), b),
        grid=(2, nblk),
        in_specs=[
            pl.BlockSpec((_MAIN_BLOCK, d), lambda p, i: (i, 0)),
            pl.BlockSpec((1, 1, _MAIN_BLOCK),
                         lambda p, i: (i * p, 0, 0)),
            pl.BlockSpec(memory_space=pltpu.MemorySpace.HBM),
            pl.BlockSpec(memory_space=pltpu.MemorySpace.SMEM),
            pl.BlockSpec((h_dim, d), full),
            pl.BlockSpec((h_dim, d), full),
            pl.BlockSpec((1, h_dim), full),
            pl.BlockSpec((1, h_dim), full),
            pl.BlockSpec((1, d), full),
            pl.BlockSpec((1, d), full),
        ],
        out_specs=pl.BlockSpec((b, d), full),
        out_shape=jax.ShapeDtypeStruct((b, d), jnp.float32),
        scratch_shapes=[
            pltpu.VMEM((8, d), jnp.float32),
            pltpu.VMEM((b + _WIN, h_dim), jnp.bfloat16),
            pltpu.VMEM((b + _WIN, 2 * d), jnp.float32),
            pltpu.VMEM((h_dim, d), jnp.bfloat16),
            pltpu.VMEM((b, d), jnp.float32),
            pltpu.SemaphoreType.DMA,
        ],
    )(feat_m, gid, feat_m, ln2, Wu, Wv,
      bv.reshape(1, -1), We, gamma.reshape(1, -1), beta.reshape(1, -1))
    return out
